# baseline (speedup 1.0000x reference)
import functools

import jax
import jax.numpy as jnp
from jax import lax
from jax.experimental import pallas as pl
from jax.experimental.pallas import tpu as pltpu
from jax.experimental.pallas import tpu_sc as plsc

_MAXLEN = 200
_EMBED = 64
_BATCH = 1024
_NC, _NS, _L = 2, 16, 16
_NW = _NC * _NS
_SEQ_PER_W = _BATCH // _NW
_IDX_W = 100
_CHUNKS_PER_SEQ = _MAXLEN // _IDX_W


@functools.partial(
    pl.kernel,
    out_type=jax.ShapeDtypeStruct((_BATCH * _MAXLEN, _EMBED), jnp.float32),
    mesh=plsc.VectorSubcoreMesh(core_axis_name="c", subcore_axis_name="s"),
    scratch_types=[
        pltpu.VMEM((_SEQ_PER_W * _CHUNKS_PER_SEQ, _IDX_W), jnp.int32),
        pltpu.VMEM((_MAXLEN, _EMBED), jnp.float32),
        pltpu.VMEM((_MAXLEN, _EMBED), jnp.float32),
        pltpu.SemaphoreType.DMA,
    ],
    compiler_params=pltpu.CompilerParams(use_tc_tiling_on_sc=False),
)
def _emb_kernel(x_hbm, table_hbm, pos_hbm, out_hbm, idx_v, pos_v, buf, sem):
    wid = lax.axis_index("s") * _NC + lax.axis_index("c")
    pltpu.sync_copy(x_hbm.at[wid], idx_v)
    pltpu.sync_copy(pos_hbm, pos_v)

    def seq_body(s, carry):
        pltpu.async_copy(
            table_hbm.at[idx_v.at[2 * s]], buf.at[pl.ds(0, _IDX_W)], sem
        ).wait()
        pltpu.async_copy(
            table_hbm.at[idx_v.at[2 * s + 1]], buf.at[pl.ds(_IDX_W, _IDX_W)], sem
        ).wait()

        def add_body(r, c):
            for k in range(_EMBED // _L):
                sl = pl.ds(k * _L, _L)
                buf[r, sl] = buf[r, sl] + pos_v[r, sl]
            return c

        lax.fori_loop(0, _MAXLEN, add_body, 0)
        seq = wid * _SEQ_PER_W + s
        pltpu.sync_copy(buf, out_hbm.at[pl.ds(seq * _MAXLEN, _MAXLEN)])
        return carry

    lax.fori_loop(0, _SEQ_PER_W, seq_body, 0)


def kernel(x, token_table, pos_table):
    x_r = x.reshape(_NW, _SEQ_PER_W * _CHUNKS_PER_SEQ, _IDX_W)
    out = _emb_kernel(x_r, token_table, pos_table)
    return out.reshape(_BATCH, _MAXLEN, _EMBED)


# speedup vs baseline: 1.0664x; 1.0664x over previous
import functools

import jax
import jax.numpy as jnp
from jax import lax
from jax.experimental import pallas as pl
from jax.experimental.pallas import tpu as pltpu
from jax.experimental.pallas import tpu_sc as plsc

_MAXLEN = 200
_EMBED = 64
_BATCH = 1024
_NC, _NS, _L = 2, 16, 16
_NW = _NC * _NS
_SEQ_PER_W = _BATCH // _NW
_IDX_W = 100
_SEQ_PER_STEP = 2
_ROWS = _SEQ_PER_STEP * _MAXLEN
_GPS = _ROWS // _IDX_W
_NSTEPS = _SEQ_PER_W // _SEQ_PER_STEP
_IDX_ROWS = _SEQ_PER_W * _MAXLEN // _IDX_W


@functools.partial(
    pl.kernel,
    out_type=jax.ShapeDtypeStruct((_BATCH * _MAXLEN, _EMBED), jnp.float32),
    mesh=plsc.VectorSubcoreMesh(core_axis_name="c", subcore_axis_name="s"),
    scratch_types=[
        pltpu.VMEM((_IDX_ROWS, _IDX_W), jnp.int32),
        pltpu.VMEM((_ROWS, _EMBED), jnp.float32),
        pltpu.VMEM((_ROWS, _EMBED), jnp.float32),
        pltpu.VMEM((_ROWS, _EMBED), jnp.float32),
        pltpu.SemaphoreType.DMA,
        pltpu.SemaphoreType.DMA,
        pltpu.SemaphoreType.DMA,
        pltpu.SemaphoreType.DMA,
    ],
    compiler_params=pltpu.CompilerParams(use_tc_tiling_on_sc=False),
)
def _emb_kernel(x_hbm, table_hbm, pos_hbm, out_hbm,
                idx_v, pos_v, buf0, buf1, gsem0, gsem1, ssem0, ssem1):
    wid = lax.axis_index("s") * _NC + lax.axis_index("c")
    pltpu.sync_copy(x_hbm.at[wid], idx_v)
    pltpu.sync_copy(pos_hbm, pos_v)

    bufs = (buf0, buf1)
    gsems = (gsem0, gsem1)
    ssems = (ssem0, ssem1)
    out_base = wid * _SEQ_PER_W * _MAXLEN

    def issue_gathers(g, buf, gsem):
        for j in range(_GPS):
            pltpu.async_copy(
                table_hbm.at[idx_v.at[_GPS * g + j]],
                buf.at[pl.ds(j * _IDX_W, _IDX_W)],
                gsem,
            )

    def wait_gathers(buf, gsem):
        for j in range(_GPS):
            pltpu.make_async_copy(
                table_hbm.at[pl.ds(0, _IDX_W)],
                buf.at[pl.ds(j * _IDX_W, _IDX_W)],
                gsem,
            ).wait()

    def wait_store(buf, ssem):
        pltpu.make_async_copy(buf, out_hbm.at[pl.ds(0, _ROWS)], ssem).wait()

    issue_gathers(0, buf0, gsem0)

    def outer(i, carry):
        for b in range(2):
            g = 2 * i + b
            cur, nxt = bufs[b], bufs[1 - b]

            @pl.when(g >= 1)
            def _():
                wait_store(nxt, ssems[1 - b])

            @pl.when(g + 1 < _NSTEPS)
            def _():
                issue_gathers(g + 1, nxt, gsems[1 - b])

            wait_gathers(cur, gsems[b])

            @plsc.parallel_loop(0, _ROWS, step=1, unroll=8)
            def _(r):
                for k in range(_EMBED // _L):
                    sl = pl.ds(k * _L, _L)
                    cur[r, sl] = cur[r, sl] + pos_v[r, sl]

            pltpu.async_copy(
                cur, out_hbm.at[pl.ds(out_base + g * _ROWS, _ROWS)], ssems[b]
            )
        return carry

    lax.fori_loop(0, _NSTEPS // 2, outer, 0)
    wait_store(buf1, ssem1)


def kernel(x, token_table, pos_table):
    x_r = x.reshape(_NW, _IDX_ROWS, _IDX_W)
    pos2 = jnp.concatenate([pos_table] * _SEQ_PER_STEP, axis=0)
    out = _emb_kernel(x_r, token_table, pos2)
    return out.reshape(_BATCH, _MAXLEN, _EMBED)


# speedup vs baseline: 1.1489x; 1.0773x over previous
import functools

import jax
import jax.numpy as jnp
from jax import lax
from jax.experimental import pallas as pl
from jax.experimental.pallas import tpu as pltpu
from jax.experimental.pallas import tpu_sc as plsc

_MAXLEN = 200
_EMBED = 64
_BATCH = 1024
_NC, _NS, _L = 2, 16, 16
_NW = _NC * _NS
_SEQ_PER_W = _BATCH // _NW
_IDX_W = 100
_ROWS = _MAXLEN
_GPS = _ROWS // _IDX_W
_NSTEPS = _SEQ_PER_W
_IDX_ROWS = _SEQ_PER_W * _MAXLEN // _IDX_W


@functools.partial(
    pl.kernel,
    out_type=jax.ShapeDtypeStruct((_BATCH * _MAXLEN, _EMBED), jnp.float32),
    mesh=plsc.VectorSubcoreMesh(core_axis_name="c", subcore_axis_name="s"),
    scratch_types=[
        pltpu.VMEM((_IDX_ROWS, _IDX_W), jnp.int32),
        pltpu.VMEM((_ROWS, _EMBED), jnp.float32),
        pltpu.VMEM((_ROWS, 128), jnp.float32),
        pltpu.VMEM((_ROWS, 128), jnp.float32),
        pltpu.VMEM((_ROWS, _EMBED), jnp.float32),
        pltpu.VMEM((_ROWS, _EMBED), jnp.float32),
        pltpu.SemaphoreType.DMA,
        pltpu.SemaphoreType.DMA,
        pltpu.SemaphoreType.DMA,
        pltpu.SemaphoreType.DMA,
    ],
    compiler_params=pltpu.CompilerParams(use_tc_tiling_on_sc=False),
)
def _emb_kernel(x_hbm, table_hbm, pos_hbm, out_hbm,
                idx_v, pos_v, buf0, buf1, st0, st1,
                gsem0, gsem1, ssem0, ssem1):
    wid = lax.axis_index("s") * _NC + lax.axis_index("c")
    pltpu.sync_copy(x_hbm.at[wid], idx_v)
    pltpu.sync_copy(pos_hbm, pos_v)

    bufs = (buf0, buf1)
    sts = (st0, st1)
    gsems = (gsem0, gsem1)
    ssems = (ssem0, ssem1)
    out_base = wid * _SEQ_PER_W * _MAXLEN

    def issue_gathers(g, buf, gsem):
        for j in range(_GPS):
            pltpu.async_copy(
                table_hbm.at[idx_v.at[_GPS * g + j]],
                buf.at[pl.ds(j * _IDX_W, _IDX_W)],
                gsem,
            )

    def wait_gathers(buf, gsem):
        for j in range(_GPS):
            pltpu.make_async_copy(
                table_hbm.at[pl.ds(0, _IDX_W)],
                buf.at[pl.ds(j * _IDX_W, _IDX_W)],
                gsem,
            ).wait()

    def wait_store(st, ssem):
        pltpu.make_async_copy(st, out_hbm.at[pl.ds(0, _ROWS)], ssem).wait()

    issue_gathers(0, buf0, gsem0)

    def outer(i, carry):
        for b in range(2):
            g = 2 * i + b
            cur, nxt = bufs[b], bufs[1 - b]
            st = sts[b]

            @pl.when(g + 1 < _NSTEPS)
            def _():
                issue_gathers(g + 1, nxt, gsems[1 - b])

            @pl.when(g >= 2)
            def _():
                wait_store(st, ssems[b])

            wait_gathers(cur, gsems[b])

            @plsc.parallel_loop(0, _ROWS, step=1, unroll=8)
            def _(r):
                for k in range(_EMBED // _L):
                    sl = pl.ds(k * _L, _L)
                    st[r, sl] = cur[r, sl] + pos_v[r, sl]

            pltpu.async_copy(
                st, out_hbm.at[pl.ds(out_base + g * _ROWS, _ROWS)], ssems[b]
            )
        return carry

    lax.fori_loop(0, _NSTEPS // 2, outer, 0)
    wait_store(st0, ssem0)
    wait_store(st1, ssem1)


def kernel(x, token_table, pos_table):
    x_r = x.reshape(_NW, _IDX_ROWS, _IDX_W)
    table128 = jnp.pad(token_table, ((0, 0), (0, 128 - _EMBED)))
    out = _emb_kernel(x_r, table128, pos_table)
    return out.reshape(_BATCH, _MAXLEN, _EMBED)


# speedup vs baseline: 1.7257x; 1.5020x over previous
import functools

import jax
import jax.numpy as jnp
from jax import lax
from jax.experimental import pallas as pl
from jax.experimental.pallas import tpu as pltpu
from jax.experimental.pallas import tpu_sc as plsc

_MAXLEN = 200
_EMBED = 64
_BATCH = 1024
_VOCAB = 1000000
_NC, _NS, _L = 2, 16, 16
_NW = _NC * _NS
_TOK_PER_W = _BATCH * _MAXLEN // _NW
_STEP = 128
_NSTEPS = _TOK_PER_W // _STEP
_SPAN = 4096
_RROWS = 244 * 2048 + 576


def _repack_body(tt_ref, out_ref):
    blk = tt_ref[...]
    out_ref[:, 0:_EMBED] = blk[:, 0:_SPAN // 2].T
    out_ref[:, _EMBED:128] = blk[:, _SPAN // 2:_SPAN].T


_repack_kernel = pl.pallas_call(
    _repack_body,
    out_shape=jax.ShapeDtypeStruct((_RROWS, 128), jnp.float32),
    grid=(pl.cdiv(_VOCAB, _SPAN),),
    in_specs=[pl.BlockSpec((_EMBED, _SPAN), lambda i: (0, i))],
    out_specs=pl.BlockSpec((_SPAN // 2, 128), lambda i: (i, 0)),
)


@functools.partial(
    pl.kernel,
    out_type=jax.ShapeDtypeStruct((_BATCH * _MAXLEN, _EMBED), jnp.float32),
    mesh=plsc.VectorSubcoreMesh(core_axis_name="c", subcore_axis_name="s"),
    scratch_types=[
        pltpu.VMEM((_NSTEPS, _STEP), jnp.int32),
        pltpu.VMEM((_NSTEPS, _STEP), jnp.int32),
        pltpu.VMEM((_MAXLEN, _EMBED), jnp.float32),
        pltpu.VMEM((_STEP, 128), jnp.float32),
        pltpu.VMEM((_STEP, 128), jnp.float32),
        pltpu.VMEM((_STEP, _EMBED), jnp.float32),
        pltpu.VMEM((_STEP, _EMBED), jnp.float32),
        pltpu.SemaphoreType.DMA,
        pltpu.SemaphoreType.DMA,
        pltpu.SemaphoreType.DMA,
        pltpu.SemaphoreType.DMA,
    ],
    compiler_params=pltpu.CompilerParams(use_tc_tiling_on_sc=True),
)
def _emb_kernel(x_hbm, table_hbm, pos_hbm, out_hbm,
                idx_v, idx_g, pos_v, buf0, buf1, st0, st1,
                gsem0, gsem1, ssem0, ssem1):
    wid = lax.axis_index("s") * _NC + lax.axis_index("c")
    pltpu.sync_copy(x_hbm.at[wid], idx_v)
    pltpu.sync_copy(pos_hbm, pos_v)

    @plsc.parallel_loop(0, _NSTEPS, step=1, unroll=2)
    def _(j):
        for k in range(_STEP // _L):
            sl = pl.ds(k * _L, _L)
            v = idx_v[j, sl]
            idx_g[j, sl] = jnp.left_shift(jnp.right_shift(v, 12), 11) + (v & 2047)

    bufs = (buf0, buf1)
    sts = (st0, st1)
    gsems = (gsem0, gsem1)
    ssems = (ssem0, ssem1)
    out_base = wid * _TOK_PER_W

    def issue_gather(g, buf, gsem):
        pltpu.async_copy(table_hbm.at[idx_g.at[g]], buf, gsem)

    def wait_gather(g, buf, gsem):
        pltpu.make_async_copy(table_hbm.at[idx_g.at[g]], buf, gsem).wait()

    def wait_store(st, ssem):
        pltpu.make_async_copy(st, out_hbm.at[pl.ds(0, _STEP)], ssem).wait()

    issue_gather(0, buf0, gsem0)

    def outer(i, carry):
        for b in range(2):
            g = 2 * i + b
            cur, nxt = bufs[b], bufs[1 - b]
            st = sts[b]

            @pl.when(g + 1 < _NSTEPS)
            def _():
                issue_gather(g + 1, nxt, gsems[1 - b])

            @pl.when(g >= 2)
            def _():
                wait_store(st, ssems[b])

            wait_gather(g, cur, gsems[b])

            base_t = lax.rem(g * _STEP, _MAXLEN)

            @plsc.parallel_loop(0, _STEP, step=1, unroll=8)
            def _(r):
                t = base_t + r
                t = jnp.where(t >= _MAXLEN, t - _MAXLEN, t)
                vr = idx_v[g, pl.ds(r, 1)]
                half = (jnp.right_shift(vr[0], 11) & 1) * _EMBED
                for k in range(_EMBED // _L):
                    st[r, pl.ds(k * _L, _L)] = (
                        cur[r, pl.ds(half + k * _L, _L)]
                        + pos_v[t, pl.ds(k * _L, _L)]
                    )

            pltpu.async_copy(
                st, out_hbm.at[pl.ds(out_base + g * _STEP, _STEP)], ssems[b]
            )
        return carry

    lax.fori_loop(0, _NSTEPS // 2, outer, 0)
    wait_store(st0, ssem0)
    wait_store(st1, ssem1)


def kernel(x, token_table, pos_table):
    x_r = x.reshape(_NW, _NSTEPS, _STEP)
    repacked = _repack_kernel(token_table.T)
    out = _emb_kernel(x_r, repacked, pos_table)
    return out.reshape(_BATCH, _MAXLEN, _EMBED)


# speedup vs baseline: 1.9968x; 1.1571x over previous
import functools

import jax
import jax.numpy as jnp
from jax import lax
from jax.experimental import pallas as pl
from jax.experimental.pallas import tpu as pltpu
from jax.experimental.pallas import tpu_sc as plsc

_MAXLEN = 200
_EMBED = 64
_BATCH = 1024
_VOCAB = 1000000
_NC, _NS, _L = 2, 16, 16
_NW = _NC * _NS
_TOK_PER_W = _BATCH * _MAXLEN // _NW
_STEP = 128
_NSTEPS = _TOK_PER_W // _STEP
_SPAN = 8192
_RROWS = 122 * 4096 + 576


def _repack_body(tt_ref, out_ref):
    out_ref[:, 0:_EMBED] = tt_ref[:, 0:_SPAN // 2].T
    out_ref[:, _EMBED:128] = tt_ref[:, _SPAN // 2:_SPAN].T


_repack_kernel = pl.pallas_call(
    _repack_body,
    out_shape=jax.ShapeDtypeStruct((_RROWS, 128), jnp.float32),
    grid=(pl.cdiv(_VOCAB, _SPAN),),
    in_specs=[pl.BlockSpec((_EMBED, _SPAN), lambda i: (0, i))],
    out_specs=pl.BlockSpec((_SPAN // 2, 128), lambda i: (i, 0)),
)


@functools.partial(
    pl.kernel,
    out_type=jax.ShapeDtypeStruct((_BATCH * _MAXLEN, _EMBED), jnp.float32),
    mesh=plsc.VectorSubcoreMesh(core_axis_name="c", subcore_axis_name="s"),
    scratch_types=[
        pltpu.VMEM((_NSTEPS, _STEP), jnp.int32),
        pltpu.VMEM((_NSTEPS, _STEP), jnp.int32),
        pltpu.VMEM((_MAXLEN, _EMBED), jnp.float32),
        pltpu.VMEM((_STEP, 128), jnp.float32),
        pltpu.VMEM((_STEP, 128), jnp.float32),
        pltpu.VMEM((_STEP, _EMBED), jnp.float32),
        pltpu.VMEM((_STEP, _EMBED), jnp.float32),
        pltpu.SemaphoreType.DMA,
        pltpu.SemaphoreType.DMA,
        pltpu.SemaphoreType.DMA,
        pltpu.SemaphoreType.DMA,
    ],
    compiler_params=pltpu.CompilerParams(use_tc_tiling_on_sc=True),
)
def _emb_kernel(x_hbm, table_hbm, pos_hbm, out_hbm,
                idx_v, idx_g, pos_v, buf0, buf1, st0, st1,
                gsem0, gsem1, ssem0, ssem1):
    wid = lax.axis_index("s") * _NC + lax.axis_index("c")
    pltpu.sync_copy(x_hbm.at[wid], idx_v)
    pltpu.sync_copy(pos_hbm, pos_v)

    @plsc.parallel_loop(0, _NSTEPS, step=1, unroll=2)
    def _(j):
        for k in range(_STEP // _L):
            sl = pl.ds(k * _L, _L)
            v = idx_v[j, sl]
            idx_g[j, sl] = jnp.left_shift(jnp.right_shift(v, 13), 12) + (v & 4095)

    bufs = (buf0, buf1)
    sts = (st0, st1)
    gsems = (gsem0, gsem1)
    ssems = (ssem0, ssem1)
    out_base = wid * _TOK_PER_W

    def issue_gather(g, buf, gsem):
        pltpu.async_copy(table_hbm.at[idx_g.at[g]], buf, gsem)

    def wait_gather(g, buf, gsem):
        pltpu.make_async_copy(table_hbm.at[idx_g.at[g]], buf, gsem).wait()

    def wait_store(st, ssem):
        pltpu.make_async_copy(st, out_hbm.at[pl.ds(0, _STEP)], ssem).wait()

    issue_gather(0, buf0, gsem0)

    def outer(i, carry):
        for b in range(2):
            g = 2 * i + b
            cur, nxt = bufs[b], bufs[1 - b]
            st = sts[b]

            @pl.when(g + 1 < _NSTEPS)
            def _():
                issue_gather(g + 1, nxt, gsems[1 - b])

            @pl.when(g >= 2)
            def _():
                wait_store(st, ssems[b])

            wait_gather(g, cur, gsems[b])

            base_t = lax.rem(g * _STEP, _MAXLEN)

            @plsc.parallel_loop(0, _STEP, step=1, unroll=8)
            def _(r):
                t = base_t + r
                t = jnp.where(t >= _MAXLEN, t - _MAXLEN, t)
                vr = idx_v[g, pl.ds(r, 1)]
                half = (jnp.right_shift(vr[0], 12) & 1) * _EMBED
                for k in range(_EMBED // _L):
                    st[r, pl.ds(k * _L, _L)] = (
                        cur[r, pl.ds(half + k * _L, _L)]
                        + pos_v[t, pl.ds(k * _L, _L)]
                    )

            pltpu.async_copy(
                st, out_hbm.at[pl.ds(out_base + g * _STEP, _STEP)], ssems[b]
            )
        return carry

    lax.fori_loop(0, _NSTEPS // 2, outer, 0)
    wait_store(st0, ssem0)
    wait_store(st1, ssem1)


def kernel(x, token_table, pos_table):
    x_r = x.reshape(_NW, _NSTEPS, _STEP)
    repacked = _repack_kernel(token_table.T)
    out = _emb_kernel(x_r, repacked, pos_table)
    return out.reshape(_BATCH, _MAXLEN, _EMBED)
